# baseline (speedup 1.0000x reference)
import functools

import jax
import jax.numpy as jnp
from jax import lax
from jax.experimental import pallas as pl
from jax.experimental.pallas import tpu as pltpu
from jax.experimental.pallas import tpu_sc as plsc

_LANES = 128


def _sc_gather(pos_flat, table_flat, n, v, d):
    nw = 32
    per_w = n // nw
    ch = 1024
    n_chunks = per_w // ch
    mesh = plsc.VectorSubcoreMesh(core_axis_name="c", subcore_axis_name="s")

    @functools.partial(
        pl.kernel,
        mesh=mesh,
        compiler_params=pltpu.CompilerParams(needs_layout_passes=False),
        out_type=jax.ShapeDtypeStruct((n * d,), jnp.float32),
        scratch_types=[
            pltpu.VMEM((v * d,), jnp.float32),
            pltpu.VMEM((ch,), jnp.int32),
            pltpu.VMEM((ch * d,), jnp.float32),
        ],
    )
    def gather_kernel(pos_hbm, table_hbm, out_hbm, table_v, idx_v, rows_v):
        wid = lax.axis_index("s") * 2 + lax.axis_index("c")
        base = wid * per_w
        pltpu.sync_copy(table_hbm, table_v)

        def chunk_body(s, carry):
            tok0 = base + s * ch

            pltpu.sync_copy(pos_hbm.at[pl.ds(tok0, ch)], idx_v)
            lane = lax.iota(jnp.int32, 16)

            @plsc.parallel_loop(0, ch // 16, unroll=2)
            def grp_body(g):
                for j in range(16):
                    t = g * 16 + j
                    pj = plsc.load_gather(idx_v, [jnp.full((16,), t, jnp.int32)])
                    src = pj * d + lane
                    rows_v[pl.ds(t * d, 16)] = plsc.load_gather(table_v, [src])
                    rows_v[pl.ds(t * d + 16, 16)] = plsc.load_gather(
                        table_v, [src + 16]
                    )
            pltpu.sync_copy(rows_v, out_hbm.at[pl.ds(tok0 * d, ch * d)])
            return carry

        lax.fori_loop(0, n_chunks, chunk_body, 0)

    return gather_kernel(pos_flat, table_flat)


def _tc_combine(x, posemb_flat, wt, b2d):
    bsz, seq, di = x.shape
    de = wt.shape[1]
    bb = 16
    toks = bb * seq

    def body(x_ref, pe_ref, wt_ref, b_ref, o_ref):
        xb = x_ref[...].reshape(toks, di)
        acc = jnp.dot(xb, wt_ref[...], preferred_element_type=jnp.float32)
        o_ref[...] = (acc + b_ref[...] + pe_ref[...]).reshape(bb, seq, de)

    return pl.pallas_call(
        body,
        grid=(bsz // bb,),
        in_specs=[
            pl.BlockSpec((bb, seq, di), lambda i: (i, 0, 0)),
            pl.BlockSpec((toks, de), lambda i: (i, 0)),
            pl.BlockSpec((di, de), lambda i: (0, 0)),
            pl.BlockSpec((1, de), lambda i: (0, 0)),
        ],
        out_specs=pl.BlockSpec((bb, seq, de), lambda i: (i, 0, 0)),
        out_shape=jax.ShapeDtypeStruct((bsz, seq, de), jnp.float32),
    )(x, posemb_flat.reshape(-1, de), wt, b2d)


def kernel(x, positions, pos_table, W, b):
    bsz, seq, di = x.shape
    de = W.shape[0]
    n = bsz * seq
    pos_flat = positions.reshape(n).astype(jnp.int32)
    posemb = _sc_gather(pos_flat, pos_table.reshape(-1), n, pos_table.shape[0], de)
    return _tc_combine(x, posemb, W.T, b.reshape(1, de))


# speedup vs baseline: 1.1082x; 1.1082x over previous
import functools

import jax
import jax.numpy as jnp
from jax import lax
from jax.experimental import pallas as pl
from jax.experimental.pallas import tpu as pltpu
from jax.experimental.pallas import tpu_sc as plsc

_LANES = 128


def _sc_gather(pos_flat, table_flat, n, v, d):
    nw = 32
    per_w = n // nw
    ch = 1024
    n_chunks = per_w // ch
    mesh = plsc.VectorSubcoreMesh(core_axis_name="c", subcore_axis_name="s")

    @functools.partial(
        pl.kernel,
        mesh=mesh,
        compiler_params=pltpu.CompilerParams(needs_layout_passes=False),
        out_type=jax.ShapeDtypeStruct((n * d,), jnp.float32),
        scratch_types=[
            pltpu.VMEM((v * d,), jnp.float32),
            pltpu.VMEM((ch,), jnp.int32),
            pltpu.VMEM((ch * d,), jnp.float32),
        ],
    )
    def gather_kernel(pos_hbm, table_hbm, out_hbm, table_v, idx_v, rows_v):
        wid = lax.axis_index("s") * 2 + lax.axis_index("c")
        base = wid * per_w
        pltpu.sync_copy(table_hbm, table_v)

        def chunk_body(s, carry):
            tok0 = base + s * ch

            pltpu.sync_copy(pos_hbm.at[pl.ds(tok0, ch)], idx_v)
            lane = lax.iota(jnp.int32, 16)

            @plsc.parallel_loop(0, ch // 16, unroll=2)
            def grp_body(g):
                for j in range(16):
                    t = g * 16 + j
                    pj = plsc.load_gather(idx_v, [jnp.full((16,), t, jnp.int32)])
                    src = pj * d + lane
                    rows_v[pl.ds(t * d, 16)] = plsc.load_gather(table_v, [src])
                    rows_v[pl.ds(t * d + 16, 16)] = plsc.load_gather(
                        table_v, [src + 16]
                    )
            pltpu.sync_copy(rows_v, out_hbm.at[pl.ds(tok0 * d, ch * d)])
            return carry

        lax.fori_loop(0, n_chunks, chunk_body, 0)

    return gather_kernel(pos_flat, table_flat)


def _tc_combine(x, posemb_flat, wt, b2d):
    bsz, seq, di = x.shape
    de = wt.shape[1]
    bb = 16
    toks = bb * seq

    def body(x_ref, pe_ref, wt_ref, b_ref, o_ref):
        xb = x_ref[...].reshape(toks, di)
        acc = jnp.dot(xb, wt_ref[...], preferred_element_type=jnp.float32)
        o_ref[...] = acc + b_ref[...] + pe_ref[...]

    out = pl.pallas_call(
        body,
        grid=(bsz // bb,),
        in_specs=[
            pl.BlockSpec((bb, seq, di), lambda i: (i, 0, 0)),
            pl.BlockSpec((toks, de), lambda i: (i, 0)),
            pl.BlockSpec((di, de), lambda i: (0, 0)),
            pl.BlockSpec((1, de), lambda i: (0, 0)),
        ],
        out_specs=pl.BlockSpec((toks, de), lambda i: (i, 0)),
        out_shape=jax.ShapeDtypeStruct((bsz * seq, de), jnp.float32),
    )(x, posemb_flat.reshape(-1, de), wt, b2d)
    return out.reshape(bsz, seq, de)


def kernel(x, positions, pos_table, W, b):
    bsz, seq, di = x.shape
    de = W.shape[0]
    n = bsz * seq
    pos_flat = positions.reshape(n).astype(jnp.int32)
    posemb = _sc_gather(pos_flat, pos_table.reshape(-1), n, pos_table.shape[0], de)
    return _tc_combine(x, posemb, W.T, b.reshape(1, de))


# speedup vs baseline: 5.1415x; 4.6396x over previous
import functools

import jax
import jax.numpy as jnp
from jax import lax
from jax.experimental import pallas as pl
from jax.experimental.pallas import tpu as pltpu
from jax.experimental.pallas import tpu_sc as plsc


def _sc_gather_t(pos_t_flat, table_t_flat, seq, bsz, v, de):
    nw = 32
    per_b = bsz // nw
    mesh = plsc.VectorSubcoreMesh(core_axis_name="c", subcore_axis_name="s")

    @functools.partial(
        pl.kernel,
        mesh=mesh,
        compiler_params=pltpu.CompilerParams(needs_layout_passes=False),
        out_type=jax.ShapeDtypeStruct((seq * de, bsz), jnp.float32),
        scratch_types=[
            pltpu.VMEM((v * de,), jnp.float32),
            pltpu.VMEM((per_b,), jnp.int32),
            pltpu.VMEM((per_b,), jnp.int32),
            pltpu.VMEM((de, per_b), jnp.float32),
            pltpu.VMEM((de, per_b), jnp.float32),
            pltpu.SemaphoreType.DMA,
            pltpu.SemaphoreType.DMA,
            pltpu.SemaphoreType.DMA,
            pltpu.SemaphoreType.DMA,
        ],
    )
    def gather_kernel(pos_hbm, tab_hbm, out_hbm, tab_v, idx0, idx1,
                      rows0, rows1, si0, si1, so0, so1):
        wid = lax.axis_index("s") * 2 + lax.axis_index("c")
        b0 = wid * per_b
        pltpu.sync_copy(tab_hbm, tab_v)
        pltpu.async_copy(pos_hbm.at[pl.ds(b0, per_b)], idx0, si0)

        def pair_body(i, carry):
            for p in (0, 1):
                s = 2 * i + p
                idx_v = (idx0, idx1)[p]
                rows_v = (rows0, rows1)[p]
                si = (si0, si1)[p]
                so = (so0, so1)[p]
                idx_n = (idx1, idx0)[p]
                si_n = (si1, si0)[p]

                @pl.when(s + 1 < seq)
                def _():
                    pltpu.async_copy(
                        pos_hbm.at[pl.ds((s + 1) * bsz + b0, per_b)], idx_n, si_n
                    )

                pltpu.make_async_copy(
                    pos_hbm.at[pl.ds(s * bsz + b0, per_b)], idx_v, si
                ).wait()

                @pl.when(s >= 2)
                def _():
                    pltpu.make_async_copy(
                        rows_v, out_hbm.at[pl.ds(0, de), pl.ds(b0, per_b)], so
                    ).wait()

                @plsc.parallel_loop(0, per_b // 16, unroll=2)
                def grp(g):
                    idx16 = idx_v[pl.ds(g * 16, 16)]
                    for e in range(de):
                        val = plsc.load_gather(tab_v, [idx16 + e * v])
                        rows_v[e, pl.ds(g * 16, 16)] = val

                pltpu.async_copy(
                    rows_v, out_hbm.at[pl.ds(s * de, de), pl.ds(b0, per_b)], so
                )
            return carry

        lax.fori_loop(0, seq // 2, pair_body, 0)
        pltpu.make_async_copy(
            rows0, out_hbm.at[pl.ds(0, de), pl.ds(b0, per_b)], so0
        ).wait()
        pltpu.make_async_copy(
            rows1, out_hbm.at[pl.ds(0, de), pl.ds(b0, per_b)], so1
        ).wait()

    return gather_kernel(pos_t_flat, table_t_flat)


def _tc_combine_t(x_t2, pe_t2, w, b128, seq, bsz, di, de):
    bl = 4096

    def body(x_ref, pe_ref, w_ref, b_ref, o_ref):
        acc = jnp.dot(w_ref[...], x_ref[...], preferred_element_type=jnp.float32)
        o_ref[...] = acc + b_ref[:, 0:1] + pe_ref[...]

    return pl.pallas_call(
        body,
        grid=(seq, bsz // bl),
        in_specs=[
            pl.BlockSpec((di, bl), lambda s, l: (s, l)),
            pl.BlockSpec((de, bl), lambda s, l: (s, l)),
            pl.BlockSpec((de, di), lambda s, l: (0, 0)),
            pl.BlockSpec((de, 128), lambda s, l: (0, 0)),
        ],
        out_specs=pl.BlockSpec((de, bl), lambda s, l: (s, l)),
        out_shape=jax.ShapeDtypeStruct((seq * de, bsz), jnp.float32),
    )(x_t2, pe_t2, w, b128)


def kernel(x, positions, pos_table, W, b):
    bsz, seq, di = x.shape
    v, de = pos_table.shape

    x_t2 = x.transpose(1, 2, 0).reshape(seq * di, bsz)
    pos_t_flat = positions.T.astype(jnp.int32).reshape(seq * bsz)
    table_t_flat = pos_table.T.reshape(de * v)
    b128 = jnp.broadcast_to(b.reshape(de, 1), (de, 128))

    pe_t2 = _sc_gather_t(pos_t_flat, table_t_flat, seq, bsz, v, de)
    out_t2 = _tc_combine_t(x_t2, pe_t2, W, b128, seq, bsz, di, de)
    return out_t2.reshape(seq, de, bsz).transpose(2, 0, 1)
